# Initial kernel scaffold; baseline (speedup 1.0000x reference)
import jax
import jax.numpy as jnp
from jax.experimental import pallas as pl


def kernel(x, seg, tok_embed, pos_embed, seg_embed):
    raise NotImplementedError("write your pallas kernel here")


# speedup vs baseline: 1.8515x; 1.8515x over previous
import jax
import jax.numpy as jnp
from jax import lax
from jax.experimental import pallas as pl
from jax.experimental.pallas import tpu as pltpu
from jax.experimental.pallas import tpu_sc as plsc

D = 128
LANES = 16
DCHUNKS = D // LANES


def _embed_body(x_hbm, seg_hbm, tok_hbm, pos_hbm, segtab_hbm, out_hbm,
                idx_v, seg_v, rows_v, pos_v, segtab_v, sem):
    nc = 2
    c = lax.axis_index("c")
    s = lax.axis_index("s")
    wid = s * nc + c
    chunk = rows_v.shape[0]
    seq = pos_hbm.shape[0]
    base = wid * chunk
    posbase = lax.rem(base, seq)

    pltpu.sync_copy(x_hbm.at[pl.ds(wid * 2, 2)], idx_v)
    pltpu.sync_copy(seg_hbm.at[pl.ds(base, chunk)], seg_v)

    cp0 = pltpu.async_copy(tok_hbm.at[idx_v.at[0]], rows_v.at[pl.ds(0, 128)], sem)
    cp1 = pltpu.async_copy(tok_hbm.at[idx_v.at[1]], rows_v.at[pl.ds(128, 128)], sem)

    pltpu.sync_copy(pos_hbm.at[pl.ds(posbase, chunk)], pos_v)
    pltpu.sync_copy(segtab_hbm, segtab_v)
    cp0.wait()
    cp1.wait()

    seg0 = [segtab_v[0, pl.ds(j * LANES, LANES)] for j in range(DCHUNKS)]
    dif = [segtab_v[1, pl.ds(j * LANES, LANES)] - seg0[j] for j in range(DCHUNKS)]

    def grp_body(g, carry):
        sv = seg_v[pl.ds(g * LANES, LANES)].astype(jnp.float32)

        def tok_body(t, c2):
            i = g * LANES + t
            sf = jnp.take(sv, jnp.full((LANES,), t, jnp.int32),
                          mode="fill")
            for j in range(DCHUNKS):
                sl = pl.ds(j * LANES, LANES)
                rows_v[i, sl] = rows_v[i, sl] + pos_v[i, sl] + seg0[j] + sf * dif[j]
            return c2

        return lax.fori_loop(0, LANES, tok_body, carry)

    lax.fori_loop(0, chunk // LANES, grp_body, 0)

    pltpu.sync_copy(rows_v, out_hbm.at[pl.ds(base, chunk)])


def kernel(x, seg, tok_embed, pos_embed, seg_embed):
    batch, seq = x.shape
    n = batch * seq
    nw = 32
    chunk = n // nw

    xf = x.astype(jnp.int32).reshape(nw * 2, chunk // 2)
    segf = seg.astype(jnp.int32).reshape(n)

    mesh = plsc.VectorSubcoreMesh(core_axis_name="c", subcore_axis_name="s")
    out = pl.kernel(
        _embed_body,
        out_type=jax.ShapeDtypeStruct((n, D), jnp.float32),
        mesh=mesh,
        scratch_types=[
            pltpu.VMEM((2, chunk // 2), jnp.int32),
            pltpu.VMEM((chunk,), jnp.int32),
            pltpu.VMEM((chunk, D), jnp.float32),
            pltpu.VMEM((chunk, D), jnp.float32),
            pltpu.VMEM((2, D), jnp.float32),
            pltpu.SemaphoreType.DMA,
        ],
    )(xf, segf, tok_embed, pos_embed, seg_embed)
    return out.reshape(batch, seq, D)


# speedup vs baseline: 2.5239x; 1.3631x over previous
import jax
import jax.numpy as jnp
from jax import lax
from jax.experimental import pallas as pl
from jax.experimental.pallas import tpu as pltpu
from jax.experimental.pallas import tpu_sc as plsc

D = 128
LANES = 16
DCHUNKS = D // LANES
NSUB = 4


def _embed_body(x_hbm, seg_hbm, tok_hbm, pos_hbm, segtab_hbm, out_hbm,
                idx_v, seg_v, rows_v, pos_v, segtab_v,
                tok_sems, pos_sems, out_sem):
    nc = 2
    c = lax.axis_index("c")
    s = lax.axis_index("s")
    wid = s * nc + c
    chunk = rows_v.shape[0]
    sub = chunk // NSUB
    seq = pos_hbm.shape[0]
    base = wid * chunk
    posbase = lax.rem(base, seq)

    pltpu.sync_copy(x_hbm.at[pl.ds(wid * NSUB, NSUB)], idx_v)
    pltpu.sync_copy(seg_hbm.at[pl.ds(base, chunk)], seg_v)
    pltpu.sync_copy(segtab_hbm, segtab_v)

    copies = []
    for k in range(NSUB):
        sl = pl.ds(k * sub, sub)
        tok_cp = pltpu.async_copy(tok_hbm.at[idx_v.at[k]], rows_v.at[sl],
                                  tok_sems[k])
        pos_cp = pltpu.async_copy(pos_hbm.at[pl.ds(posbase + k * sub, sub)],
                                  pos_v.at[sl], pos_sems[k])
        copies.append((tok_cp, pos_cp))

    seg0 = [segtab_v[0, pl.ds(j * LANES, LANES)] for j in range(DCHUNKS)]
    dif = [segtab_v[1, pl.ds(j * LANES, LANES)] - seg0[j] for j in range(DCHUNKS)]

    def grp_body(g, carry):
        sv = seg_v[pl.ds(g * LANES, LANES)].astype(jnp.float32)

        def tok_body(t, c2):
            i = g * LANES + t
            sf = jnp.take(sv, jnp.full((LANES,), t, jnp.int32),
                          mode="fill")
            toks = [rows_v[i, pl.ds(j * LANES, LANES)] for j in range(DCHUNKS)]
            poss = [pos_v[i, pl.ds(j * LANES, LANES)] for j in range(DCHUNKS)]
            for j in range(DCHUNKS):
                rows_v[i, pl.ds(j * LANES, LANES)] = (
                    toks[j] + poss[j] + (seg0[j] + sf * dif[j]))
            return c2

        return lax.fori_loop(0, LANES, tok_body, carry)

    out_cps = []
    gps = sub // LANES
    for k in range(NSUB):
        copies[k][0].wait()
        copies[k][1].wait()
        lax.fori_loop(k * gps, (k + 1) * gps, grp_body, 0)
        sl = pl.ds(k * sub, sub)
        out_cps.append(pltpu.async_copy(rows_v.at[sl],
                                        out_hbm.at[pl.ds(base + k * sub, sub)],
                                        out_sem))
    for cp in out_cps:
        cp.wait()


def kernel(x, seg, tok_embed, pos_embed, seg_embed):
    batch, seq = x.shape
    n = batch * seq
    nw = 32
    chunk = n // nw
    sub = chunk // NSUB

    xf = x.astype(jnp.int32).reshape(nw * NSUB, sub)
    segf = seg.astype(jnp.int32).reshape(n)

    mesh = plsc.VectorSubcoreMesh(core_axis_name="c", subcore_axis_name="s")
    out = pl.kernel(
        _embed_body,
        out_type=jax.ShapeDtypeStruct((n, D), jnp.float32),
        mesh=mesh,
        scratch_types=[
            pltpu.VMEM((NSUB, sub), jnp.int32),
            pltpu.VMEM((chunk,), jnp.int32),
            pltpu.VMEM((chunk, D), jnp.float32),
            pltpu.VMEM((chunk, D), jnp.float32),
            pltpu.VMEM((2, D), jnp.float32),
            [pltpu.SemaphoreType.DMA] * NSUB,
            [pltpu.SemaphoreType.DMA] * NSUB,
            pltpu.SemaphoreType.DMA,
        ],
    )(xf, segf, tok_embed, pos_embed, seg_embed)
    return out.reshape(batch, seq, D)


# speedup vs baseline: 2.6916x; 1.0665x over previous
import jax
import jax.numpy as jnp
from jax import lax
from jax.experimental import pallas as pl
from jax.experimental.pallas import tpu as pltpu
from jax.experimental.pallas import tpu_sc as plsc

D = 128
LANES = 16
DCHUNKS = D // LANES
NSUB = 4


def _embed_body(x_hbm, seg_hbm, tok_hbm, pos_hbm, segtab_hbm, out_hbm,
                idx_v, seg_v, rows_v, pos_v, segtab_v,
                tok_sems, pos_sems, out_sem):
    nc = 2
    c = lax.axis_index("c")
    s = lax.axis_index("s")
    wid = s * nc + c
    chunk = rows_v.shape[0]
    sub = chunk // NSUB
    seq = pos_hbm.shape[0]
    base = wid * chunk
    bb = lax.div(base, seq)
    off = lax.rem(base, seq)

    pltpu.sync_copy(seg_hbm.at[bb, pl.ds(off, chunk)], seg_v)
    pltpu.sync_copy(segtab_hbm, segtab_v)

    copies = []
    for k in range(NSUB):
        sl = pl.ds(k * sub, sub)
        pltpu.sync_copy(x_hbm.at[bb, pl.ds(off + k * sub, sub)], idx_v.at[k])
        tok_cp = pltpu.async_copy(tok_hbm.at[idx_v.at[k]], rows_v.at[sl],
                                  tok_sems[k])
        pos_cp = pltpu.async_copy(pos_hbm.at[pl.ds(off + k * sub, sub)],
                                  pos_v.at[sl], pos_sems[k])
        copies.append((tok_cp, pos_cp))

    seg0 = [segtab_v[0, pl.ds(j * LANES, LANES)] for j in range(DCHUNKS)]
    dif = [segtab_v[1, pl.ds(j * LANES, LANES)] - seg0[j] for j in range(DCHUNKS)]

    def grp_body(g, carry):
        sv = seg_v[pl.ds(g * LANES, LANES)].astype(jnp.float32)

        def tok_body(t, c2):
            i = g * LANES + t
            sf = jnp.take(sv, jnp.full((LANES,), t, jnp.int32),
                          mode="fill")
            toks = [rows_v[i, pl.ds(j * LANES, LANES)] for j in range(DCHUNKS)]
            poss = [pos_v[i, pl.ds(j * LANES, LANES)] for j in range(DCHUNKS)]
            for j in range(DCHUNKS):
                rows_v[i, pl.ds(j * LANES, LANES)] = (
                    toks[j] + poss[j] + (seg0[j] + sf * dif[j]))
            return c2

        return lax.fori_loop(0, LANES, tok_body, carry)

    out_cps = []
    gps = sub // LANES
    for k in range(NSUB):
        copies[k][0].wait()
        copies[k][1].wait()
        lax.fori_loop(k * gps, (k + 1) * gps, grp_body, 0)
        sl = pl.ds(k * sub, sub)
        out_cps.append(pltpu.async_copy(
            rows_v.at[sl],
            out_hbm.at[bb, pl.ds(off + k * sub, sub)],
            out_sem))
    for cp in out_cps:
        cp.wait()


def kernel(x, seg, tok_embed, pos_embed, seg_embed):
    batch, seq = x.shape
    n = batch * seq
    nw = 32
    chunk = n // nw
    sub = chunk // NSUB

    mesh = plsc.VectorSubcoreMesh(core_axis_name="c", subcore_axis_name="s")
    out = pl.kernel(
        _embed_body,
        out_type=jax.ShapeDtypeStruct((batch, seq, D), jnp.float32),
        mesh=mesh,
        scratch_types=[
            pltpu.VMEM((NSUB, sub), jnp.int32),
            pltpu.VMEM((chunk,), jnp.int32),
            pltpu.VMEM((chunk, D), jnp.float32),
            pltpu.VMEM((chunk, D), jnp.float32),
            pltpu.VMEM((2, D), jnp.float32),
            [pltpu.SemaphoreType.DMA] * NSUB,
            [pltpu.SemaphoreType.DMA] * NSUB,
            pltpu.SemaphoreType.DMA,
        ],
    )(x.astype(jnp.int32), seg.astype(jnp.int32), tok_embed, pos_embed,
      seg_embed)
    return out
